# baseline (speedup 1.0000x reference)
import functools

import jax
import jax.numpy as jnp
from jax import lax
from jax.experimental import pallas as pl
from jax.experimental.pallas import tpu as pltpu
from jax.experimental.pallas import tpu_sc as plsc

_B, _T = 16384, 4096
_SWAPW = 16
_NW = 32
_RPW = _B // _NW
_C = 8
_CHUNKS = _RPW // _C
_NBUF = 2


def _sc_copy_swap_body(tokens_hbm, out_hbm, b0, b1, si0, si1, so0, so1):
    wid = lax.axis_index("s") * 2 + lax.axis_index("c")
    base = wid * _RPW
    bufs = (b0, b1)
    sin = (si0, si1)
    sout = (so0, so1)

    iot = lax.iota(jnp.int32, _SWAPW)
    perm = jnp.where(iot == 0, 1, jnp.where(iot == 1, 0, iot))
    dnums = lax.GatherDimensionNumbers(
        offset_dims=(), collapsed_slice_dims=(0,), start_index_map=(0,))

    def swap_head(buf):
        for r in range(_C):
            v = buf[r, pl.ds(0, _SWAPW)]
            buf[r, pl.ds(0, _SWAPW)] = lax.gather(
                v, perm[:, None], dimension_numbers=dnums, slice_sizes=(1,),
                mode=lax.GatherScatterMode.PROMISE_IN_BOUNDS)

    def rows_of(g):
        return pl.ds(base + g * _C, _C)

    for b in range(_NBUF):
        pltpu.async_copy(tokens_hbm.at[rows_of(b)], bufs[b], sin[b])

    def ring_iter(i, refill):
        for b in range(_NBUF):
            g = _NBUF * i + b
            rows = rows_of(g)
            pltpu.make_async_copy(tokens_hbm.at[rows], bufs[b], sin[b]).wait()
            swap_head(bufs[b])
            pltpu.async_copy(bufs[b], out_hbm.at[rows], sout[b])
        if refill:
            for b in range(_NBUF):
                g2 = _NBUF * i + b + _NBUF
                rows2 = rows_of(g2)
                pltpu.make_async_copy(bufs[b], out_hbm.at[rows2], sout[b]).wait()
                pltpu.async_copy(tokens_hbm.at[rows2], bufs[b], sin[b])
        return 0

    n_iter = _CHUNKS // _NBUF
    lax.fori_loop(0, n_iter - 1, lambda i, c: ring_iter(i, True), 0)
    ring_iter(n_iter - 1, False)
    for b in range(_NBUF):
        pltpu.make_async_copy(bufs[b], out_hbm.at[rows_of(b)], sout[b]).wait()


@functools.cache
def _sc_copy_swap():
    return pl.kernel(
        _sc_copy_swap_body,
        out_type=jax.ShapeDtypeStruct((_B, _T), jnp.float32),
        mesh=plsc.VectorSubcoreMesh(core_axis_name="c", subcore_axis_name="s"),
        scratch_types=[
            pltpu.VMEM((_C, _T), jnp.float32),
            pltpu.VMEM((_C, _T), jnp.float32),
            pltpu.SemaphoreType.DMA,
            pltpu.SemaphoreType.DMA,
            pltpu.SemaphoreType.DMA,
            pltpu.SemaphoreType.DMA,
        ],
    )


def kernel(tokens):
    return _sc_copy_swap()(tokens)


# speedup vs baseline: 1.0151x; 1.0151x over previous
import functools

import jax
import jax.numpy as jnp
from jax import lax
from jax.experimental import pallas as pl
from jax.experimental.pallas import tpu as pltpu
from jax.experimental.pallas import tpu_sc as plsc

_B, _T = 16384, 4096
_SWAPW = 16
_NW = 32
_RPW = _B // _NW
_C = 8
_CHUNKS = _RPW // _C
_NBUF = 3


def _sc_copy_swap_body(tokens_hbm, out_hbm, b0, b1, b2, si0, si1, si2,
                       so0, so1, so2):
    wid = lax.axis_index("s") * 2 + lax.axis_index("c")
    base = wid * _RPW
    bufs = (b0, b1, b2)
    sin = (si0, si1, si2)
    sout = (so0, so1, so2)

    iot = lax.iota(jnp.int32, _SWAPW)
    perm = jnp.where(iot == 0, 1, jnp.where(iot == 1, 0, iot))
    dnums = lax.GatherDimensionNumbers(
        offset_dims=(), collapsed_slice_dims=(0,), start_index_map=(0,))

    def swap_head(buf):
        for r in range(_C):
            v = buf[r, pl.ds(0, _SWAPW)]
            buf[r, pl.ds(0, _SWAPW)] = lax.gather(
                v, perm[:, None], dimension_numbers=dnums, slice_sizes=(1,),
                mode=lax.GatherScatterMode.PROMISE_IN_BOUNDS)

    def rows_of(g):
        return pl.ds(base + g * _C, _C)

    for b in range(_NBUF):
        pltpu.async_copy(tokens_hbm.at[rows_of(b)], bufs[b], sin[b])

    def ring_iter(i, refill):
        for b in range(_NBUF):
            g = _NBUF * i + b
            rows = rows_of(g)
            pltpu.make_async_copy(tokens_hbm.at[rows], bufs[b], sin[b]).wait()
            swap_head(bufs[b])
            pltpu.async_copy(bufs[b], out_hbm.at[rows], sout[b])
        if refill:
            for b in range(_NBUF):
                g2 = _NBUF * i + b + _NBUF
                rows2 = rows_of(g2)
                pltpu.make_async_copy(bufs[b], out_hbm.at[rows2], sout[b]).wait()
                pltpu.async_copy(tokens_hbm.at[rows2], bufs[b], sin[b])
        return 0

    n_iter = _CHUNKS // _NBUF
    rem = _CHUNKS - n_iter * _NBUF
    lax.fori_loop(0, n_iter - 1, lambda i, c: ring_iter(i, True), 0)

    for b in range(_NBUF):
        g = _NBUF * (n_iter - 1) + b
        rows = rows_of(g)
        pltpu.make_async_copy(tokens_hbm.at[rows], bufs[b], sin[b]).wait()
        swap_head(bufs[b])
        pltpu.async_copy(bufs[b], out_hbm.at[rows], sout[b])
    for b in range(rem):
        g2 = _NBUF * n_iter + b
        rows2 = rows_of(g2)
        pltpu.make_async_copy(bufs[b], out_hbm.at[rows2], sout[b]).wait()
        pltpu.async_copy(tokens_hbm.at[rows2], bufs[b], sin[b])
        pltpu.make_async_copy(tokens_hbm.at[rows2], bufs[b], sin[b]).wait()
        swap_head(bufs[b])
        pltpu.async_copy(bufs[b], out_hbm.at[rows2], sout[b])
    for b in range(_NBUF):
        pltpu.make_async_copy(bufs[b], out_hbm.at[rows_of(b)], sout[b]).wait()


@functools.cache
def _sc_copy_swap():
    return pl.kernel(
        _sc_copy_swap_body,
        out_type=jax.ShapeDtypeStruct((_B, _T), jnp.float32),
        mesh=plsc.VectorSubcoreMesh(core_axis_name="c", subcore_axis_name="s"),
        scratch_types=(
            [pltpu.VMEM((_C, _T), jnp.float32)] * _NBUF
            + [pltpu.SemaphoreType.DMA] * (2 * _NBUF)
        ),
    )


def kernel(tokens):
    return _sc_copy_swap()(tokens)


# speedup vs baseline: 1.0778x; 1.0617x over previous
import functools

import jax
import jax.numpy as jnp
from jax import lax
from jax.experimental import pallas as pl
from jax.experimental.pallas import tpu as pltpu
from jax.experimental.pallas import tpu_sc as plsc

_B, _T = 16384, 4096
_HEAD = 128
_SWAPW = 16
_NW = 32
_RPW = _B // _NW
_GR = 512


_SC_C = 128
_SC_CHUNKS = _RPW // _SC_C


def _sc_head_swap_body(tokens_hbm, head_hbm, b0, b1, si0, si1, so0, so1):
    wid = lax.axis_index("s") * 2 + lax.axis_index("c")
    base = wid * _RPW
    bufs = (b0, b1)
    sin = (si0, si1)
    sout = (so0, so1)

    iot = lax.iota(jnp.int32, _SWAPW)
    perm = jnp.where(iot == 0, 1, jnp.where(iot == 1, 0, iot))
    dnums = lax.GatherDimensionNumbers(
        offset_dims=(), collapsed_slice_dims=(0,), start_index_map=(0,))

    def swap_chunk(buf):
        def step8(i, carry):
            for k in range(8):
                r = i * 8 + k
                v = buf[r, pl.ds(0, _SWAPW)]
                buf[r, pl.ds(0, _SWAPW)] = lax.gather(
                    v, perm[:, None], dimension_numbers=dnums,
                    slice_sizes=(1,),
                    mode=lax.GatherScatterMode.PROMISE_IN_BOUNDS)
            return carry

        lax.fori_loop(0, _SC_C // 8, step8, 0)

    def src(g):
        return tokens_hbm.at[pl.ds(base + g * _SC_C, _SC_C), pl.ds(0, _HEAD)]

    def dst(g):
        return head_hbm.at[pl.ds(base + g * _SC_C, _SC_C), pl.ds(0, _HEAD)]

    for b in range(2):
        pltpu.async_copy(src(b), bufs[b], sin[b])
    for g in range(_SC_CHUNKS):
        b = g % 2
        pltpu.make_async_copy(src(g), bufs[b], sin[b]).wait()
        swap_chunk(bufs[b])
        pltpu.async_copy(bufs[b], dst(g), sout[b])
        if g + 2 < _SC_CHUNKS:
            pltpu.make_async_copy(bufs[b], dst(g), sout[b]).wait()
            pltpu.async_copy(src(g + 2), bufs[b], sin[b])
    for g in (_SC_CHUNKS - 2, _SC_CHUNKS - 1):
        pltpu.make_async_copy(bufs[g % 2], dst(g), sout[g % 2]).wait()


@functools.cache
def _sc_head_swap():
    return pl.kernel(
        _sc_head_swap_body,
        out_type=jax.ShapeDtypeStruct((_B, _HEAD), jnp.float32),
        mesh=plsc.VectorSubcoreMesh(core_axis_name="c", subcore_axis_name="s"),
        scratch_types=(
            [pltpu.VMEM((_SC_C, _HEAD), jnp.float32)] * 2
            + [pltpu.SemaphoreType.DMA] * 4
        ),
    )


def _tc_copy_body(tok_ref, head_ref, out_ref):
    out_ref[...] = tok_ref[...]
    out_ref[:, 0:_HEAD] = head_ref[...]


@functools.cache
def _tc_copy():
    return pl.pallas_call(
        _tc_copy_body,
        grid=(_B // _GR,),
        in_specs=[
            pl.BlockSpec((_GR, _T), lambda i: (i, 0)),
            pl.BlockSpec((_GR, _HEAD), lambda i: (i, 0)),
        ],
        out_specs=pl.BlockSpec((_GR, _T), lambda i: (i, 0)),
        out_shape=jax.ShapeDtypeStruct((_B, _T), jnp.float32),
        compiler_params=pltpu.CompilerParams(
            dimension_semantics=("arbitrary",),
        ),
    )


def kernel(tokens):
    head = _sc_head_swap()(tokens)
    return _tc_copy()(tokens, head)
